# baseline (speedup 1.0000x reference)
import jax
import jax.numpy as jnp
from jax.experimental import pallas as pl
from jax.experimental.pallas import tpu as pltpu


def _body(bp_ref, rt_ref, rw_ref, out_ref):
    out_ref[0, 0] = bp_ref[0, 0] + rt_ref[0, 0] + rw_ref[0, 0]


@jax.jit
def _probe(bpi, rti, rw32):
    return pl.pallas_call(
        _body,
        out_shape=jax.ShapeDtypeStruct((1, 1), jnp.float32),
        out_specs=pl.BlockSpec(memory_space=pltpu.SMEM),
    )(bpi, rti, rw32)


def kernel(bbox_pred, reg_target, reg_weight):
    bpi = jnp.transpose(bbox_pred, (0, 2, 3, 1)).reshape(625, 128)
    rti = reg_target.reshape(625, 128)
    rw32 = reg_weight.reshape(625, 32)
    return _probe(bpi, rti, rw32)[0, 0]


# speedup vs baseline: 2.0426x; 2.0426x over previous
import jax
import jax.numpy as jnp
from jax.experimental import pallas as pl
from jax.experimental.pallas import tpu as pltpu

B = 32
S = 625


def _body(pl_r, pt_r, pr_r, pb_r, tl_r, tt_r, tr_r, tb_r, rw_r, out_ref):
    p_l, p_t, p_r, p_b = pl_r[...], pt_r[...], pr_r[...], pb_r[...]
    t_l, t_t, t_r, t_b = tl_r[...], tt_r[...], tr_r[...], tb_r[...]
    rw = rw_r[...]

    t_area = (t_l + t_r) * (t_t + t_b)
    p_area = (p_l + p_r) * (p_t + p_b)
    w_i = jnp.minimum(p_l, t_l) + jnp.minimum(p_r, t_r)
    h_i = jnp.minimum(p_b, t_b) + jnp.minimum(p_t, t_t)
    a_i = w_i * h_i
    a_u = t_area + p_area - a_i
    lg = jnp.log((a_i + 1.0) / (a_u + 1.0))

    m = (rw > 0.0).astype(jnp.float32)
    s = jnp.sum(lg * m)
    c = jnp.sum(m)
    out_ref[0, 0] = -s / jnp.maximum(c, 1.0)


@jax.jit
def _iou_loss(p_ch, t_ch, rw):
    return pl.pallas_call(
        _body,
        out_shape=jax.ShapeDtypeStruct((1, 1), jnp.float32),
        out_specs=pl.BlockSpec(memory_space=pltpu.SMEM),
    )(*p_ch, *t_ch, rw)


def kernel(bbox_pred, reg_target, reg_weight):
    p_ch = [bbox_pred[:, c].reshape(B, S) for c in range(4)]
    t_ch = [reg_target[..., c].reshape(B, S) for c in range(4)]
    rw = reg_weight.reshape(B, S)
    return _iou_loss(p_ch, t_ch, rw)[0, 0]


# speedup vs baseline: 2.8403x; 1.3905x over previous
import jax
import jax.numpy as jnp
from jax.experimental import pallas as pl
from jax.experimental.pallas import tpu as pltpu

B = 32
S = 625


def _body(x_ref, out_ref):
    p_l, p_t, p_r, p_b = x_ref[0], x_ref[1], x_ref[2], x_ref[3]
    t_l, t_t, t_r, t_b = x_ref[4], x_ref[5], x_ref[6], x_ref[7]
    rw = x_ref[8]

    t_area = (t_l + t_r) * (t_t + t_b)
    p_area = (p_l + p_r) * (p_t + p_b)
    w_i = jnp.minimum(p_l, t_l) + jnp.minimum(p_r, t_r)
    h_i = jnp.minimum(p_b, t_b) + jnp.minimum(p_t, t_t)
    a_i = w_i * h_i
    a_u = t_area + p_area - a_i
    lg = jnp.log((a_i + 1.0) / (a_u + 1.0))

    m = (rw > 0.0).astype(jnp.float32)
    s = jnp.sum(lg * m)
    c = jnp.sum(m)
    out_ref[0, 0] = -s / jnp.maximum(c, 1.0)


@jax.jit
def _iou_loss(x):
    return pl.pallas_call(
        _body,
        out_shape=jax.ShapeDtypeStruct((1, 1), jnp.float32),
        out_specs=pl.BlockSpec(memory_space=pltpu.SMEM),
    )(x)


def kernel(bbox_pred, reg_target, reg_weight):
    slabs = (
        [bbox_pred[:, c].reshape(B, S) for c in range(4)]
        + [reg_target[..., c].reshape(B, S) for c in range(4)]
        + [reg_weight.reshape(B, S)]
    )
    return _iou_loss(jnp.stack(slabs, axis=0))[0, 0]


# speedup vs baseline: 9.6334x; 3.3916x over previous
import jax
import jax.numpy as jnp
from jax import lax
from jax.experimental import pallas as pl
from jax.experimental.pallas import tpu as pltpu


def _body(bp_ref, rt_ref, rw_ref, out_ref):
    p = bp_ref[...].reshape(625, 128)
    t = rt_ref[...].reshape(625, 128)
    w = rw_ref[...]

    def phase_sum(x):
        return x + jnp.roll(x, -64, axis=1)

    sp = phase_sum(p)
    st = phase_sum(t)
    sm = phase_sum(jnp.minimum(p, t))
    p_area = sp * jnp.roll(sp, -32, axis=1)
    t_area = st * jnp.roll(st, -32, axis=1)
    a_i = sm * jnp.roll(sm, -32, axis=1)
    a_u = t_area + p_area - a_i
    ratio = (a_i + 1.0) / jnp.maximum(a_u + 1.0, 1e-6)
    lg = jnp.log(jnp.maximum(ratio, 1e-30))

    lg0 = lg.reshape(25, 25, 128)[:, :, :32]
    m = (w > 0.0).astype(jnp.float32)
    m_t = jnp.transpose(m, (0, 2, 1))
    s = jnp.sum(lg0 * m_t)
    c = jnp.sum(m)
    out_ref[0, 0] = -s / jnp.maximum(c, 1.0)


@jax.jit
def _iou_loss(bpt, rtt, rwj):
    return pl.pallas_call(
        _body,
        out_shape=jax.ShapeDtypeStruct((1, 1), jnp.float32),
        out_specs=pl.BlockSpec(memory_space=pltpu.SMEM),
    )(bpt, rtt, rwj)


def kernel(bbox_pred, reg_target, reg_weight):
    bpt = jnp.transpose(bbox_pred, (2, 3, 1, 0))
    rtt = jnp.transpose(reg_target, (1, 2, 3, 0))
    rwj = jnp.transpose(reg_weight, (1, 0, 2))
    return _iou_loss(bpt, rtt, rwj)[0, 0]
